# baseline (speedup 1.0000x reference)
import functools
import jax
import jax.numpy as jnp
from jax import lax
from jax.experimental import pallas as pl
from jax.experimental.pallas import tpu as pltpu
from jax.experimental.pallas import tpu_sc as plsc

DIM = 64
CHUNK = 512
L = 16


def _sc_body(xt_hbm, out_hbm, vbuf, obuf):
    wid = lax.axis_index("s") * 2 + lax.axis_index("c")
    b0 = wid * CHUNK

    def s_step(s, carry):
        pltpu.sync_copy(xt_hbm.at[s, :, pl.ds(b0, CHUNK)], vbuf)

        def g_step(g, carry2):
            base = g * L
            acc = jnp.zeros((L,), jnp.float32)
            for j in range(DIM):
                e = vbuf[j, pl.ds(base, L)]
                acc = acc + e * e
            u = jnp.maximum(acc * 0.01, 1e-14)
            sq = jnp.full((L,), 1.0, jnp.float32)
            for _ in range(12):
                sq = 0.5 * (sq + u / sq)
            e2 = jnp.exp(-2.0 * sq)
            t = (1.0 - e2) / (1.0 + e2)
            f_big = 0.1 * t / sq
            f_small = 0.1 * (1.0 - u * (1.0 / 3.0) + u * u * (2.0 / 15.0))
            f = jnp.where(u < 2.5e-3, f_small, f_big)
            for j in range(DIM):
                obuf[j, pl.ds(base, L)] = vbuf[j, pl.ds(base, L)] * f
            return carry2

        lax.fori_loop(0, CHUNK // L, g_step, 0)
        pltpu.sync_copy(obuf, out_hbm.at[s, :, pl.ds(b0, CHUNK)])
        return carry

    lax.fori_loop(0, xt_hbm.shape[0], s_step, 0)


def kernel(x):
    b, s, d = x.shape
    xt = jnp.transpose(x, (1, 2, 0))
    run = functools.partial(
        pl.kernel,
        out_type=jax.ShapeDtypeStruct((s, d, b), jnp.float32),
        mesh=plsc.VectorSubcoreMesh(core_axis_name="c", subcore_axis_name="s"),
        scratch_types=[
            pltpu.VMEM((DIM, CHUNK), jnp.float32),
            pltpu.VMEM((DIM, CHUNK), jnp.float32),
        ],
    )(_sc_body)
    out_t = run(xt)
    return jnp.transpose(out_t, (2, 0, 1))


# speedup vs baseline: 5.9117x; 5.9117x over previous
import jax
import jax.numpy as jnp
from jax.experimental import pallas as pl
from jax.experimental.pallas import tpu as pltpu


def _expmap_body(x_ref, o_ref):
    x = x_ref[...]
    n2 = jnp.sum(x * x, axis=1, keepdims=True) * 0.01
    n2 = jnp.maximum(n2, 1e-14)
    r = jax.lax.rsqrt(n2)
    n = n2 * r
    t = jnp.tanh(n)
    o_ref[...] = x * (0.1 * (t * r))


def kernel(x):
    b, s, d = x.shape
    xt = jnp.transpose(x, (1, 2, 0))
    out_t = pl.pallas_call(
        _expmap_body,
        grid=(s // 2,),
        in_specs=[pl.BlockSpec((2, d, b), lambda i: (i, 0, 0))],
        out_specs=pl.BlockSpec((2, d, b), lambda i: (i, 0, 0)),
        out_shape=jax.ShapeDtypeStruct((s, d, b), jnp.float32),
        compiler_params=pltpu.CompilerParams(
            dimension_semantics=("arbitrary",),
        ),
    )(xt)
    return jnp.transpose(out_t, (2, 0, 1))
